# baseline (speedup 1.0000x reference)
import functools

import jax
import jax.numpy as jnp
from jax import lax
from jax.experimental import pallas as pl
from jax.experimental.pallas import tpu as pltpu
from jax.experimental.pallas import tpu_sc as plsc

NQ = 10000
NKV = 10000
E = 320000
D = 128

NC = 2
NS = 16
NW = NC * NS
L = 16
D_PER = D // NW
CHUNK = 2000


def _tc_project(q_ref, kvt_ref, w_ref, b_ref, aw_ref, ab_ref,
                sq_ref, skv_ref, kvpt_ref):
    W = w_ref[...]
    b = b_ref[...]
    aw = aw_ref[...]
    aw_q = aw[:D]
    aw_kv = aw[D:]
    kvpt = jnp.dot(W, kvt_ref[...], preferred_element_type=jnp.float32)
    kvpt = kvpt + b[:, None]
    kvpt_ref[...] = kvpt
    v_q = jnp.sum(aw_q[:, None] * W, axis=0)
    const = jnp.sum(b * aw_q) + ab_ref[0, 0]
    sq_ref[...] = jnp.sum(q_ref[...] * v_q[None, :], axis=1) + const
    skv_ref[...] = jnp.sum(kvpt * aw_kv[:, None], axis=0)


_project = pl.pallas_call(
    _tc_project,
    out_shape=[
        jax.ShapeDtypeStruct((NQ,), jnp.float32),
        jax.ShapeDtypeStruct((NKV,), jnp.float32),
        jax.ShapeDtypeStruct((D, NKV), jnp.float32),
    ],
)


_mesh = plsc.VectorSubcoreMesh(core_axis_name="c", subcore_axis_name="s")


@functools.partial(
    pl.kernel,
    out_type=jax.ShapeDtypeStruct((D * NQ,), jnp.float32),
    mesh=_mesh,
    compiler_params=pltpu.CompilerParams(
        needs_layout_passes=False, disable_bounds_checks=True),
    scratch_types=[
        pltpu.VMEM((NQ,), jnp.float32),
        pltpu.VMEM((NKV,), jnp.float32),
        pltpu.VMEM((D_PER * NKV,), jnp.float32),
        pltpu.VMEM((D_PER * NQ,), jnp.float32),
        pltpu.VMEM((NQ,), jnp.float32),
        pltpu.VMEM((CHUNK,), jnp.int32),
        pltpu.VMEM((CHUNK,), jnp.int32),
    ],
)
def _sc_aggregate(qi_hbm, kvi_hbm, sq_hbm, skv_hbm, kvpt_hbm, acct_hbm,
                  sq_v, skv_v, kvp_v, acc_v, z_v, qib, kvib):
    wid = lax.axis_index("s") * NC + lax.axis_index("c")
    row0 = wid * D_PER

    pltpu.sync_copy(sq_hbm, sq_v)
    pltpu.sync_copy(skv_hbm, skv_v)
    for d in range(D_PER):
        pltpu.sync_copy(kvpt_hbm.at[pl.ds((row0 + d) * NKV, NKV)],
                        kvp_v.at[pl.ds(d * NKV, NKV)])

    zeros = jnp.zeros((L,), jnp.float32)

    @plsc.parallel_loop(0, NQ // L, unroll=8)
    def zero_body(i):
        sl = pl.ds(i * L, L)
        z_v[sl] = zeros
        for d in range(D_PER):
            acc_v[pl.ds(d * NQ + i * L, L)] = zeros

    def chunk_body(c, carry):
        off = c * CHUNK
        pltpu.sync_copy(qi_hbm.at[pl.ds(off, CHUNK)], qib)
        pltpu.sync_copy(kvi_hbm.at[pl.ds(off, CHUNK)], kvib)

        @plsc.parallel_loop(0, CHUNK // L, unroll=8)
        def group_body(g):
            sl = pl.ds(g * L, L)
            qi = qib[sl]
            kvi = kvib[sl]
            sq = plsc.load_gather(sq_v, [qi])
            skv = plsc.load_gather(skv_v, [kvi])
            e = sq + skv
            e = jnp.maximum(e, 0.2 * e)
            w = jnp.exp(e)
            plsc.addupdate_scatter(z_v, [qi], w)
            for d in range(D_PER):
                col = plsc.load_gather(kvp_v, [kvi + (d * NKV)])
                plsc.addupdate_scatter(acc_v, [qi + (d * NQ)], w * col)

        return carry

    lax.fori_loop(0, E // CHUNK, chunk_body, 0)

    @plsc.parallel_loop(0, NQ // L, unroll=8)
    def scale_body(i):
        sl = pl.ds(i * L, L)
        r = 1.0 / (z_v[sl] + 1e-10)
        for d in range(D_PER):
            fsl = pl.ds(d * NQ + i * L, L)
            acc_v[fsl] = acc_v[fsl] * r

    for d in range(D_PER):
        pltpu.sync_copy(acc_v.at[pl.ds(d * NQ, NQ)],
                        acct_hbm.at[pl.ds((row0 + d) * NQ, NQ)])


def kernel(query_nodes, key_value_nodes, edge_index, proj_w, proj_b,
           attend_w, attend_b):
    kvt = key_value_nodes.T
    ab = jnp.reshape(attend_b, (1, 1))
    sq, skv, kvpt = _project(query_nodes, kvt, proj_w, proj_b, attend_w, ab)
    acct = _sc_aggregate(edge_index[0], edge_index[1], sq, skv,
                         kvpt.reshape(-1))
    return acct.reshape(D, NQ).T


# speedup vs baseline: 1.4274x; 1.4274x over previous
import functools

import jax
import jax.numpy as jnp
from jax import lax
from jax.experimental import pallas as pl
from jax.experimental.pallas import tpu as pltpu
from jax.experimental.pallas import tpu_sc as plsc

NQ = 10000
NKV = 10000
E = 320000
D = 128

NC = 2
NS = 16
NW = NC * NS
L = 16
D_PER = D // NW
CHUNK = 3200


def _tc_project(q_ref, kvt_ref, w_ref, b_ref, aw_ref, ab_ref,
                sq_ref, skv_ref, kvpt_ref):
    W = w_ref[...]
    b = b_ref[...]
    aw = aw_ref[...]
    aw_q = aw[:D]
    aw_kv = aw[D:]
    kvpt = jnp.dot(W, kvt_ref[...], preferred_element_type=jnp.float32)
    kvpt = kvpt + b[:, None]
    kvpt_ref[...] = kvpt
    v_q = jnp.sum(aw_q[:, None] * W, axis=0)
    const = jnp.sum(b * aw_q) + ab_ref[0, 0]
    sq_ref[...] = jnp.sum(q_ref[...] * v_q[None, :], axis=1) + const
    skv_ref[...] = jnp.sum(kvpt * aw_kv[:, None], axis=0)


_project = pl.pallas_call(
    _tc_project,
    out_shape=[
        jax.ShapeDtypeStruct((NQ,), jnp.float32),
        jax.ShapeDtypeStruct((NKV,), jnp.float32),
        jax.ShapeDtypeStruct((D, NKV), jnp.float32),
    ],
)


_mesh = plsc.VectorSubcoreMesh(core_axis_name="c", subcore_axis_name="s")


@functools.partial(
    pl.kernel,
    out_type=jax.ShapeDtypeStruct((D * NQ,), jnp.float32),
    mesh=_mesh,
    compiler_params=pltpu.CompilerParams(
        needs_layout_passes=False, disable_bounds_checks=True),
    scratch_types=[
        pltpu.VMEM((NQ,), jnp.float32),
        pltpu.VMEM((NKV,), jnp.float32),
        pltpu.VMEM((D_PER * NKV,), jnp.float32),
        pltpu.VMEM((D_PER * NQ,), jnp.float32),
        pltpu.VMEM((NQ,), jnp.float32),
        pltpu.VMEM((CHUNK,), jnp.int32),
        pltpu.VMEM((CHUNK,), jnp.int32),
        pltpu.VMEM((CHUNK,), jnp.float32),
    ],
)
def _sc_aggregate(qi_hbm, kvi_hbm, sq_hbm, skv_hbm, kvpt_hbm, acct_hbm,
                  sq_v, skv_v, kvp_v, acc_v, z_v, qib, kvib, wb):
    wid = lax.axis_index("s") * NC + lax.axis_index("c")
    row0 = wid * D_PER

    pltpu.sync_copy(sq_hbm, sq_v)
    pltpu.sync_copy(skv_hbm, skv_v)
    for d in range(D_PER):
        pltpu.sync_copy(kvpt_hbm.at[pl.ds((row0 + d) * NKV, NKV)],
                        kvp_v.at[pl.ds(d * NKV, NKV)])

    zeros = jnp.zeros((L,), jnp.float32)

    @plsc.parallel_loop(0, NQ // L, unroll=8)
    def zero_body(i):
        sl = pl.ds(i * L, L)
        z_v[sl] = zeros
        for d in range(D_PER):
            acc_v[pl.ds(d * NQ + i * L, L)] = zeros

    def chunk_body(c, carry):
        off = c * CHUNK
        pltpu.sync_copy(qi_hbm.at[pl.ds(off, CHUNK)], qib)
        pltpu.sync_copy(kvi_hbm.at[pl.ds(off, CHUNK)], kvib)

        @plsc.parallel_loop(0, CHUNK // L, unroll=8)
        def weight_body(g):
            sl = pl.ds(g * L, L)
            qi = qib[sl]
            kvi = kvib[sl]
            sq = plsc.load_gather(sq_v, [qi])
            skv = plsc.load_gather(skv_v, [kvi])
            e = sq + skv
            e = jnp.maximum(e, 0.2 * e)
            w = jnp.exp(e)
            wb[sl] = w
            plsc.addupdate_scatter(z_v, [qi], w)

        @plsc.parallel_loop(0, CHUNK // L, unroll=8)
        def accum_body(g):
            sl = pl.ds(g * L, L)
            qi = qib[sl]
            kvi = kvib[sl]
            w = wb[sl]
            for d in range(D_PER):
                col = plsc.load_gather(kvp_v, [kvi + (d * NKV)])
                plsc.addupdate_scatter(acc_v, [qi + (d * NQ)], w * col)

        return carry

    lax.fori_loop(0, E // CHUNK, chunk_body, 0)

    @plsc.parallel_loop(0, NQ // L, unroll=8)
    def scale_body(i):
        sl = pl.ds(i * L, L)
        r = 1.0 / (z_v[sl] + 1e-10)
        for d in range(D_PER):
            fsl = pl.ds(d * NQ + i * L, L)
            acc_v[fsl] = acc_v[fsl] * r

    for d in range(D_PER):
        pltpu.sync_copy(acc_v.at[pl.ds(d * NQ, NQ)],
                        acct_hbm.at[pl.ds((row0 + d) * NQ, NQ)])


def kernel(query_nodes, key_value_nodes, edge_index, proj_w, proj_b,
           attend_w, attend_b):
    kvt = key_value_nodes.T
    ab = jnp.reshape(attend_b, (1, 1))
    sq, skv, kvpt = _project(query_nodes, kvt, proj_w, proj_b, attend_w, ab)
    acct = _sc_aggregate(edge_index[0], edge_index[1], sq, skv,
                         kvpt.reshape(-1))
    return acct.reshape(D, NQ).T


# speedup vs baseline: 1.8707x; 1.3106x over previous
import functools

import jax
import jax.numpy as jnp
from jax import lax
from jax.experimental import pallas as pl
from jax.experimental.pallas import tpu as pltpu
from jax.experimental.pallas import tpu_sc as plsc

NQ = 10000
NKV = 10000
E = 320000
D = 128

NC = 2
NS = 16
NW = NC * NS
L = 16
D_PER = D // NW
CHUNK = 3200


def _tc_project(q_ref, kvt_ref, w_ref, b_ref, aw_ref, ab_ref,
                sq_ref, skv_ref, kvpt_ref):
    W = w_ref[...]
    b = b_ref[...]
    aw = aw_ref[...]
    aw_q = aw[:D]
    aw_kv = aw[D:]
    kvpt = jnp.dot(W, kvt_ref[...], preferred_element_type=jnp.float32)
    kvpt = kvpt + b[:, None]
    kvpt_ref[...] = kvpt
    v_q = jnp.sum(aw_q[:, None] * W, axis=0)
    const = jnp.sum(b * aw_q) + ab_ref[0, 0]
    sq_ref[...] = jnp.sum(q_ref[...] * v_q[None, :], axis=1) + const
    skv_ref[...] = jnp.sum(kvpt * aw_kv[:, None], axis=0)


_project = pl.pallas_call(
    _tc_project,
    out_shape=[
        jax.ShapeDtypeStruct((NQ,), jnp.float32),
        jax.ShapeDtypeStruct((NKV,), jnp.float32),
        jax.ShapeDtypeStruct((D, NKV), jnp.float32),
    ],
)


_mesh = plsc.VectorSubcoreMesh(core_axis_name="c", subcore_axis_name="s")


@functools.partial(
    pl.kernel,
    out_type=jax.ShapeDtypeStruct((D * NQ,), jnp.float32),
    mesh=_mesh,
    compiler_params=pltpu.CompilerParams(
        needs_layout_passes=False, disable_bounds_checks=True),
    scratch_types=[
        pltpu.VMEM((NQ,), jnp.float32),
        pltpu.VMEM((NKV,), jnp.float32),
        pltpu.VMEM((D_PER * NKV,), jnp.float32),
        pltpu.VMEM((D_PER * NQ,), jnp.float32),
        pltpu.VMEM((NQ,), jnp.float32),
        pltpu.VMEM((2, CHUNK), jnp.int32),
        pltpu.VMEM((2, CHUNK), jnp.int32),
        pltpu.VMEM((CHUNK,), jnp.float32),
        pltpu.SemaphoreType.DMA((2,)),
        pltpu.SemaphoreType.DMA((2,)),
    ],
)
def _sc_aggregate(qi_hbm, kvi_hbm, sq_hbm, skv_hbm, kvpt_hbm, acct_hbm,
                  sq_v, skv_v, kvp_v, acc_v, z_v, qib, kvib, wb,
                  sem_q, sem_k):
    wid = lax.axis_index("s") * NC + lax.axis_index("c")
    row0 = wid * D_PER

    pltpu.sync_copy(sq_hbm, sq_v)
    pltpu.sync_copy(skv_hbm, skv_v)
    for d in range(D_PER):
        pltpu.sync_copy(kvpt_hbm.at[pl.ds((row0 + d) * NKV, NKV)],
                        kvp_v.at[pl.ds(d * NKV, NKV)])

    zeros = jnp.zeros((L,), jnp.float32)

    @plsc.parallel_loop(0, NQ // L, unroll=8)
    def zero_body(i):
        sl = pl.ds(i * L, L)
        z_v[sl] = zeros
        for d in range(D_PER):
            acc_v[pl.ds(d * NQ + i * L, L)] = zeros

    n_chunks = E // CHUNK

    pltpu.async_copy(qi_hbm.at[pl.ds(0, CHUNK)], qib.at[0], sem_q.at[0])
    pltpu.async_copy(kvi_hbm.at[pl.ds(0, CHUNK)], kvib.at[0], sem_k.at[0])

    def chunk_body(c, carry):
        buf = lax.rem(c, 2)
        nbuf = 1 - buf

        @pl.when(c + 1 < n_chunks)
        def _start_next():
            off = (c + 1) * CHUNK
            pltpu.async_copy(qi_hbm.at[pl.ds(off, CHUNK)], qib.at[nbuf],
                             sem_q.at[nbuf])
            pltpu.async_copy(kvi_hbm.at[pl.ds(off, CHUNK)], kvib.at[nbuf],
                             sem_k.at[nbuf])

        pltpu.make_async_copy(qi_hbm.at[pl.ds(0, CHUNK)], qib.at[buf],
                              sem_q.at[buf]).wait()
        pltpu.make_async_copy(kvi_hbm.at[pl.ds(0, CHUNK)], kvib.at[buf],
                              sem_k.at[buf]).wait()

        @plsc.parallel_loop(0, CHUNK // L, unroll=8)
        def weight_body(g):
            sl = pl.ds(g * L, L)
            qi = qib[buf, sl]
            kvi = kvib[buf, sl]
            sq = plsc.load_gather(sq_v, [qi])
            skv = plsc.load_gather(skv_v, [kvi])
            e = sq + skv
            e = jnp.maximum(e, 0.2 * e)
            w = jnp.exp(e)
            wb[sl] = w
            plsc.addupdate_scatter(z_v, [qi], w)

        @plsc.parallel_loop(0, CHUNK // L, unroll=8)
        def accum_body(g):
            sl = pl.ds(g * L, L)
            qi = qib[buf, sl]
            kvi = kvib[buf, sl]
            w = wb[sl]
            for d in range(D_PER):
                col = plsc.load_gather(kvp_v.at[pl.ds(d * NKV, NKV)], [kvi])
                plsc.addupdate_scatter(acc_v.at[pl.ds(d * NQ, NQ)], [qi],
                                       w * col)

        return carry

    lax.fori_loop(0, n_chunks, chunk_body, 0)

    @plsc.parallel_loop(0, NQ // L, unroll=8)
    def scale_body(i):
        sl = pl.ds(i * L, L)
        r = 1.0 / (z_v[sl] + 1e-10)
        for d in range(D_PER):
            fsl = pl.ds(d * NQ + i * L, L)
            acc_v[fsl] = acc_v[fsl] * r

    for d in range(D_PER):
        pltpu.sync_copy(acc_v.at[pl.ds(d * NQ, NQ)],
                        acct_hbm.at[pl.ds((row0 + d) * NQ, NQ)])


def kernel(query_nodes, key_value_nodes, edge_index, proj_w, proj_b,
           attend_w, attend_b):
    kvt = key_value_nodes.T
    ab = jnp.reshape(attend_b, (1, 1))
    sq, skv, kvpt = _project(query_nodes, kvt, proj_w, proj_b, attend_w, ab)
    acct = _sc_aggregate(edge_index[0], edge_index[1], sq, skv,
                         kvpt.reshape(-1))
    return acct.reshape(D, NQ).T


# speedup vs baseline: 2.0947x; 1.1197x over previous
import functools

import jax
import jax.numpy as jnp
from jax import lax
from jax.experimental import pallas as pl
from jax.experimental.pallas import tpu as pltpu
from jax.experimental.pallas import tpu_sc as plsc

NQ = 10000
NKV = 10000
E = 320000
D = 128

NC = 2
NS = 16
NW = NC * NS
L = 16
D_PER = D // NW
CHUNK = 3200


def _tc_project(q_ref, kvt_ref, w_ref, b_ref, aw_ref, ab_ref,
                sq_ref, skv_ref, kvpt_ref):
    W = w_ref[...]
    b = b_ref[...]
    aw = aw_ref[...]
    aw_q = aw[:D]
    aw_kv = aw[D:]
    kvpt = jnp.dot(W, kvt_ref[...], preferred_element_type=jnp.float32)
    kvpt = kvpt + b[:, None]
    kvpt_ref[...] = kvpt
    v_q = jnp.sum(aw_q[:, None] * W, axis=0)
    const = jnp.sum(b * aw_q) + ab_ref[0, 0]
    sq_ref[...] = jnp.sum(q_ref[...] * v_q[None, :], axis=1) + const
    skv_ref[...] = jnp.sum(kvpt * aw_kv[:, None], axis=0)


_project = pl.pallas_call(
    _tc_project,
    out_shape=[
        jax.ShapeDtypeStruct((NQ,), jnp.float32),
        jax.ShapeDtypeStruct((NKV,), jnp.float32),
        jax.ShapeDtypeStruct((D, NKV), jnp.float32),
    ],
)


_mesh = plsc.VectorSubcoreMesh(core_axis_name="c", subcore_axis_name="s")

E_PER = E // NW


@functools.partial(
    pl.kernel,
    out_type=jax.ShapeDtypeStruct((E,), jnp.float32),
    mesh=_mesh,
    compiler_params=pltpu.CompilerParams(
        needs_layout_passes=False, disable_bounds_checks=True),
    scratch_types=[
        pltpu.VMEM((NQ,), jnp.float32),
        pltpu.VMEM((NKV,), jnp.float32),
        pltpu.VMEM((E_PER,), jnp.int32),
        pltpu.VMEM((E_PER,), jnp.int32),
        pltpu.VMEM((E_PER,), jnp.float32),
    ],
)
def _sc_weights(qi_hbm, kvi_hbm, sq_hbm, skv_hbm, w_hbm,
                sq_v, skv_v, qib, kvib, wb):
    wid = lax.axis_index("s") * NC + lax.axis_index("c")
    base = wid * E_PER

    pltpu.sync_copy(sq_hbm, sq_v)
    pltpu.sync_copy(skv_hbm, skv_v)
    pltpu.sync_copy(qi_hbm.at[pl.ds(base, E_PER)], qib)
    pltpu.sync_copy(kvi_hbm.at[pl.ds(base, E_PER)], kvib)

    @plsc.parallel_loop(0, E_PER // L, unroll=8)
    def weight_body(g):
        sl = pl.ds(g * L, L)
        qi = qib[sl]
        kvi = kvib[sl]
        sq = plsc.load_gather(sq_v, [qi])
        skv = plsc.load_gather(skv_v, [kvi])
        e = sq + skv
        e = jnp.maximum(e, 0.2 * e)
        wb[sl] = jnp.exp(e)

    pltpu.sync_copy(wb, w_hbm.at[pl.ds(base, E_PER)])


@functools.partial(
    pl.kernel,
    out_type=jax.ShapeDtypeStruct((D * NQ,), jnp.float32),
    mesh=_mesh,
    compiler_params=pltpu.CompilerParams(
        needs_layout_passes=False, disable_bounds_checks=True),
    scratch_types=[
        pltpu.VMEM((D_PER * NKV,), jnp.float32),
        pltpu.VMEM((D_PER * NQ,), jnp.float32),
        pltpu.VMEM((NQ,), jnp.float32),
        pltpu.VMEM((2, CHUNK), jnp.int32),
        pltpu.VMEM((2, CHUNK), jnp.int32),
        pltpu.VMEM((2, CHUNK), jnp.float32),
        pltpu.SemaphoreType.DMA((2,)),
        pltpu.SemaphoreType.DMA((2,)),
        pltpu.SemaphoreType.DMA((2,)),
    ],
)
def _sc_aggregate(qi_hbm, kvi_hbm, w_hbm, kvpt_hbm, acct_hbm,
                  kvp_v, acc_v, z_v, qib, kvib, wib,
                  sem_q, sem_k, sem_w):
    wid = lax.axis_index("s") * NC + lax.axis_index("c")
    row0 = wid * D_PER

    for d in range(D_PER):
        pltpu.sync_copy(kvpt_hbm.at[pl.ds((row0 + d) * NKV, NKV)],
                        kvp_v.at[pl.ds(d * NKV, NKV)])

    zeros = jnp.zeros((L,), jnp.float32)

    @plsc.parallel_loop(0, NQ // L, unroll=8)
    def zero_body(i):
        sl = pl.ds(i * L, L)
        z_v[sl] = zeros
        for d in range(D_PER):
            acc_v[pl.ds(d * NQ + i * L, L)] = zeros

    n_chunks = E // CHUNK

    pltpu.async_copy(qi_hbm.at[pl.ds(0, CHUNK)], qib.at[0], sem_q.at[0])
    pltpu.async_copy(kvi_hbm.at[pl.ds(0, CHUNK)], kvib.at[0], sem_k.at[0])
    pltpu.async_copy(w_hbm.at[pl.ds(0, CHUNK)], wib.at[0], sem_w.at[0])

    def chunk_body(c, carry):
        buf = lax.rem(c, 2)
        nbuf = 1 - buf

        @pl.when(c + 1 < n_chunks)
        def _start_next():
            off = (c + 1) * CHUNK
            pltpu.async_copy(qi_hbm.at[pl.ds(off, CHUNK)], qib.at[nbuf],
                             sem_q.at[nbuf])
            pltpu.async_copy(kvi_hbm.at[pl.ds(off, CHUNK)], kvib.at[nbuf],
                             sem_k.at[nbuf])
            pltpu.async_copy(w_hbm.at[pl.ds(off, CHUNK)], wib.at[nbuf],
                             sem_w.at[nbuf])

        pltpu.make_async_copy(qi_hbm.at[pl.ds(0, CHUNK)], qib.at[buf],
                              sem_q.at[buf]).wait()
        pltpu.make_async_copy(kvi_hbm.at[pl.ds(0, CHUNK)], kvib.at[buf],
                              sem_k.at[buf]).wait()
        pltpu.make_async_copy(w_hbm.at[pl.ds(0, CHUNK)], wib.at[buf],
                              sem_w.at[buf]).wait()

        @plsc.parallel_loop(0, CHUNK // L, unroll=8)
        def accum_body(g):
            sl = pl.ds(g * L, L)
            qi = qib[buf, sl]
            kvi = kvib[buf, sl]
            w = wib[buf, sl]
            plsc.addupdate_scatter(z_v, [qi], w)
            for d in range(D_PER):
                col = plsc.load_gather(kvp_v.at[pl.ds(d * NKV, NKV)], [kvi])
                plsc.addupdate_scatter(acc_v.at[pl.ds(d * NQ, NQ)], [qi],
                                       w * col)

        return carry

    lax.fori_loop(0, n_chunks, chunk_body, 0)

    @plsc.parallel_loop(0, NQ // L, unroll=8)
    def scale_body(i):
        sl = pl.ds(i * L, L)
        r = 1.0 / (z_v[sl] + 1e-10)
        for d in range(D_PER):
            fsl = pl.ds(d * NQ + i * L, L)
            acc_v[fsl] = acc_v[fsl] * r

    for d in range(D_PER):
        pltpu.sync_copy(acc_v.at[pl.ds(d * NQ, NQ)],
                        acct_hbm.at[pl.ds((row0 + d) * NQ, NQ)])


def kernel(query_nodes, key_value_nodes, edge_index, proj_w, proj_b,
           attend_w, attend_b):
    kvt = key_value_nodes.T
    ab = jnp.reshape(attend_b, (1, 1))
    sq, skv, kvpt = _project(query_nodes, kvt, proj_w, proj_b, attend_w, ab)
    qi = edge_index[0]
    kvi = edge_index[1]
    w = _sc_weights(qi, kvi, sq, skv)
    acct = _sc_aggregate(qi, kvi, w, kvpt.reshape(-1))
    return acct.reshape(D, NQ).T


# speedup vs baseline: 2.3456x; 1.1198x over previous
import functools

import jax
import jax.numpy as jnp
from jax import lax
from jax.experimental import pallas as pl
from jax.experimental.pallas import tpu as pltpu
from jax.experimental.pallas import tpu_sc as plsc

NQ = 10000
NKV = 10000
E = 320000
D = 128

NC = 2
NS = 16
NW = NC * NS
L = 16
D_PER = D // NW
CHUNK = 6400


def _tc_project(q_ref, kvt_ref, ei_ref, w_ref, b_ref, aw_ref, ab_ref,
                sq_ref, skv_ref, kvpt_ref, qk_ref):
    W = w_ref[...]
    b = b_ref[...]
    aw = aw_ref[...]
    aw_q = aw[:D]
    aw_kv = aw[D:]
    kvpt = jnp.dot(W, kvt_ref[...], preferred_element_type=jnp.float32)
    kvpt = kvpt + b[:, None]
    kvpt_ref[...] = kvpt
    v_q = jnp.sum(aw_q[:, None] * W, axis=0)
    const = jnp.sum(b * aw_q) + ab_ref[0, 0]
    sq_ref[...] = jnp.sum(q_ref[...] * v_q[None, :], axis=1) + const
    skv_ref[...] = jnp.sum(kvpt * aw_kv[:, None], axis=0)
    qk_ref[...] = ei_ref[0] * 65536 + ei_ref[1]


_project = pl.pallas_call(
    _tc_project,
    out_shape=[
        jax.ShapeDtypeStruct((NQ,), jnp.float32),
        jax.ShapeDtypeStruct((NKV,), jnp.float32),
        jax.ShapeDtypeStruct((D, NKV), jnp.float32),
        jax.ShapeDtypeStruct((E,), jnp.int32),
    ],
)


def _unpack_qk(packed):
    qi = lax.shift_right_logical(packed, 16)
    kvi = jnp.bitwise_and(packed, 0xFFFF)
    return qi, kvi


_mesh = plsc.VectorSubcoreMesh(core_axis_name="c", subcore_axis_name="s")

E_PER = E // NW


@functools.partial(
    pl.kernel,
    out_type=jax.ShapeDtypeStruct((E,), jnp.float32),
    mesh=_mesh,
    compiler_params=pltpu.CompilerParams(
        needs_layout_passes=False, disable_bounds_checks=True),
    scratch_types=[
        pltpu.VMEM((NQ,), jnp.float32),
        pltpu.VMEM((NKV,), jnp.float32),
        pltpu.VMEM((E_PER,), jnp.int32),
        pltpu.VMEM((E_PER,), jnp.float32),
    ],
)
def _sc_weights(qk_hbm, sq_hbm, skv_hbm, w_hbm, sq_v, skv_v, qkb, wb):
    wid = lax.axis_index("s") * NC + lax.axis_index("c")
    base = wid * E_PER

    pltpu.sync_copy(sq_hbm, sq_v)
    pltpu.sync_copy(skv_hbm, skv_v)
    pltpu.sync_copy(qk_hbm.at[pl.ds(base, E_PER)], qkb)

    @plsc.parallel_loop(0, E_PER // L, unroll=8)
    def weight_body(g):
        sl = pl.ds(g * L, L)
        qi, kvi = _unpack_qk(qkb[sl])
        sq = plsc.load_gather(sq_v, [qi])
        skv = plsc.load_gather(skv_v, [kvi])
        e = sq + skv
        e = jnp.maximum(e, 0.2 * e)
        wb[sl] = jnp.exp(e)

    pltpu.sync_copy(wb, w_hbm.at[pl.ds(base, E_PER)])


@functools.partial(
    pl.kernel,
    out_type=jax.ShapeDtypeStruct((D * NQ,), jnp.float32),
    mesh=_mesh,
    compiler_params=pltpu.CompilerParams(
        needs_layout_passes=False, disable_bounds_checks=True),
    scratch_types=[
        pltpu.VMEM((D_PER * NKV,), jnp.float32),
        pltpu.VMEM((D_PER * NQ,), jnp.float32),
        pltpu.VMEM((NQ,), jnp.float32),
        pltpu.VMEM((2, CHUNK), jnp.int32),
        pltpu.VMEM((2, CHUNK), jnp.float32),
        pltpu.SemaphoreType.DMA((2,)),
        pltpu.SemaphoreType.DMA((2,)),
    ],
)
def _sc_aggregate(qk_hbm, w_hbm, kvpt_hbm, acct_hbm,
                  kvp_v, acc_v, z_v, qkib, wib, sem_q, sem_w):
    wid = lax.axis_index("s") * NC + lax.axis_index("c")
    row0 = wid * D_PER

    for d in range(D_PER):
        pltpu.sync_copy(kvpt_hbm.at[pl.ds((row0 + d) * NKV, NKV)],
                        kvp_v.at[pl.ds(d * NKV, NKV)])

    zeros = jnp.zeros((L,), jnp.float32)

    @plsc.parallel_loop(0, NQ // L, unroll=8)
    def zero_body(i):
        sl = pl.ds(i * L, L)
        z_v[sl] = zeros
        for d in range(D_PER):
            acc_v[pl.ds(d * NQ + i * L, L)] = zeros

    n_chunks = E // CHUNK

    pltpu.async_copy(qk_hbm.at[pl.ds(0, CHUNK)], qkib.at[0], sem_q.at[0])
    pltpu.async_copy(w_hbm.at[pl.ds(0, CHUNK)], wib.at[0], sem_w.at[0])

    def chunk_body(c, carry):
        buf = lax.rem(c, 2)
        nbuf = 1 - buf

        @pl.when(c + 1 < n_chunks)
        def _start_next():
            off = (c + 1) * CHUNK
            pltpu.async_copy(qk_hbm.at[pl.ds(off, CHUNK)], qkib.at[nbuf],
                             sem_q.at[nbuf])
            pltpu.async_copy(w_hbm.at[pl.ds(off, CHUNK)], wib.at[nbuf],
                             sem_w.at[nbuf])

        pltpu.make_async_copy(qk_hbm.at[pl.ds(0, CHUNK)], qkib.at[buf],
                              sem_q.at[buf]).wait()
        pltpu.make_async_copy(w_hbm.at[pl.ds(0, CHUNK)], wib.at[buf],
                              sem_w.at[buf]).wait()

        @plsc.parallel_loop(0, CHUNK // L, unroll=8)
        def accum_body(g):
            sl = pl.ds(g * L, L)
            qi, kvi = _unpack_qk(qkib[buf, sl])
            w = wib[buf, sl]
            plsc.addupdate_scatter(z_v, [qi], w)
            for d in range(D_PER):
                col = plsc.load_gather(kvp_v.at[pl.ds(d * NKV, NKV)], [kvi])
                plsc.addupdate_scatter(acc_v.at[pl.ds(d * NQ, NQ)], [qi],
                                       w * col)

        return carry

    lax.fori_loop(0, n_chunks, chunk_body, 0)

    @plsc.parallel_loop(0, NQ // L, unroll=8)
    def scale_body(i):
        sl = pl.ds(i * L, L)
        r = 1.0 / (z_v[sl] + 1e-10)
        for d in range(D_PER):
            fsl = pl.ds(d * NQ + i * L, L)
            acc_v[fsl] = acc_v[fsl] * r

    for d in range(D_PER):
        pltpu.sync_copy(acc_v.at[pl.ds(d * NQ, NQ)],
                        acct_hbm.at[pl.ds((row0 + d) * NQ, NQ)])


def kernel(query_nodes, key_value_nodes, edge_index, proj_w, proj_b,
           attend_w, attend_b):
    kvt = key_value_nodes.T
    ab = jnp.reshape(attend_b, (1, 1))
    sq, skv, kvpt, qk = _project(query_nodes, kvt, edge_index, proj_w,
                                 proj_b, attend_w, ab)
    w = _sc_weights(qk, sq, skv)
    acct = _sc_aggregate(qk, w, kvpt.reshape(-1))
    return acct.reshape(D, NQ).T


# speedup vs baseline: 2.5002x; 1.0659x over previous
import functools

import jax
import jax.numpy as jnp
from jax import lax
from jax.experimental import pallas as pl
from jax.experimental.pallas import tpu as pltpu
from jax.experimental.pallas import tpu_sc as plsc

NQ = 10000
NKV = 10000
E = 320000
D = 128

NC = 2
NS = 16
NW = NC * NS
L = 16
D_PER = D // NW
CHUNK = 6400


def _tc_project(q_ref, kvt_ref, ei_ref, we_ref, wo_ref, be_ref, bo_ref,
                awqe_ref, awqo_ref, awkve_ref, awkvo_ref, ab_ref,
                sq_ref, skv_ref, kvpp_ref, qk_ref):
    kvt = kvt_ref[...]
    kvpt_e = jnp.dot(we_ref[...], kvt, preferred_element_type=jnp.float32)
    kvpt_e = kvpt_e + be_ref[...][:, None]
    kvpt_o = jnp.dot(wo_ref[...], kvt, preferred_element_type=jnp.float32)
    kvpt_o = kvpt_o + bo_ref[...][:, None]
    lo = jax.lax.bitcast_convert_type(
        kvpt_e.astype(jnp.bfloat16), jnp.uint16).astype(jnp.int32)
    hi = jax.lax.bitcast_convert_type(
        kvpt_o.astype(jnp.bfloat16), jnp.uint16).astype(jnp.int32)
    kvpp_ref[...] = lo | (hi << 16)
    v_q = (jnp.sum(awqe_ref[...][:, None] * we_ref[...], axis=0)
           + jnp.sum(awqo_ref[...][:, None] * wo_ref[...], axis=0))
    const = (jnp.sum(be_ref[...] * awqe_ref[...])
             + jnp.sum(bo_ref[...] * awqo_ref[...]) + ab_ref[0, 0])
    sq_ref[...] = jnp.sum(q_ref[...] * v_q[None, :], axis=1) + const
    skv_ref[...] = (jnp.sum(kvpt_e * awkve_ref[...][:, None], axis=0)
                    + jnp.sum(kvpt_o * awkvo_ref[...][:, None], axis=0))
    qk_ref[...] = ei_ref[0] * 65536 + ei_ref[1]


_project = pl.pallas_call(
    _tc_project,
    out_shape=[
        jax.ShapeDtypeStruct((NQ,), jnp.float32),
        jax.ShapeDtypeStruct((NKV,), jnp.float32),
        jax.ShapeDtypeStruct((D // 2, NKV), jnp.int32),
        jax.ShapeDtypeStruct((E,), jnp.int32),
    ],
)


def _unpack_qk(packed):
    qi = lax.shift_right_logical(packed, 16)
    kvi = jnp.bitwise_and(packed, 0xFFFF)
    return qi, kvi


_mesh = plsc.VectorSubcoreMesh(core_axis_name="c", subcore_axis_name="s")

E_PER = E // NW


@functools.partial(
    pl.kernel,
    out_type=jax.ShapeDtypeStruct((E,), jnp.float32),
    mesh=_mesh,
    compiler_params=pltpu.CompilerParams(
        needs_layout_passes=False, disable_bounds_checks=True),
    scratch_types=[
        pltpu.VMEM((NQ,), jnp.float32),
        pltpu.VMEM((NKV,), jnp.float32),
        pltpu.VMEM((E_PER,), jnp.int32),
        pltpu.VMEM((E_PER,), jnp.float32),
    ],
)
def _sc_weights(qk_hbm, sq_hbm, skv_hbm, w_hbm, sq_v, skv_v, qkb, wb):
    wid = lax.axis_index("s") * NC + lax.axis_index("c")
    base = wid * E_PER

    pltpu.sync_copy(sq_hbm, sq_v)
    pltpu.sync_copy(skv_hbm, skv_v)
    pltpu.sync_copy(qk_hbm.at[pl.ds(base, E_PER)], qkb)

    @plsc.parallel_loop(0, E_PER // L, unroll=8)
    def weight_body(g):
        sl = pl.ds(g * L, L)
        qi, kvi = _unpack_qk(qkb[sl])
        sq = plsc.load_gather(sq_v, [qi])
        skv = plsc.load_gather(skv_v, [kvi])
        e = sq + skv
        e = jnp.maximum(e, 0.2 * e)
        wb[sl] = jnp.exp(e)

    pltpu.sync_copy(wb, w_hbm.at[pl.ds(base, E_PER)])


@functools.partial(
    pl.kernel,
    out_type=jax.ShapeDtypeStruct((D * NQ,), jnp.float32),
    mesh=_mesh,
    compiler_params=pltpu.CompilerParams(
        needs_layout_passes=False, disable_bounds_checks=True),
    scratch_types=[
        pltpu.VMEM((D_PER // 2 * NKV,), jnp.int32),
        pltpu.VMEM((D_PER * NQ,), jnp.float32),
        pltpu.VMEM((NQ,), jnp.float32),
        pltpu.VMEM((2, CHUNK), jnp.int32),
        pltpu.VMEM((2, CHUNK), jnp.float32),
        pltpu.SemaphoreType.DMA((2,)),
        pltpu.SemaphoreType.DMA((2,)),
    ],
)
def _sc_aggregate(qk_hbm, w_hbm, kvpt_hbm, acct_hbm,
                  kvp_v, acc_v, z_v, qkib, wib, sem_q, sem_w):
    wid = lax.axis_index("s") * NC + lax.axis_index("c")
    row0 = wid * D_PER
    pair0 = wid * (D_PER // 2)

    for j in range(D_PER // 2):
        pltpu.sync_copy(kvpt_hbm.at[pl.ds((pair0 + j) * NKV, NKV)],
                        kvp_v.at[pl.ds(j * NKV, NKV)])

    zeros = jnp.zeros((L,), jnp.float32)

    @plsc.parallel_loop(0, NQ // L, unroll=8)
    def zero_body(i):
        sl = pl.ds(i * L, L)
        z_v[sl] = zeros
        for d in range(D_PER):
            acc_v[pl.ds(d * NQ + i * L, L)] = zeros

    n_chunks = E // CHUNK

    pltpu.async_copy(qk_hbm.at[pl.ds(0, CHUNK)], qkib.at[0], sem_q.at[0])
    pltpu.async_copy(w_hbm.at[pl.ds(0, CHUNK)], wib.at[0], sem_w.at[0])

    def chunk_body(c, carry):
        buf = lax.rem(c, 2)
        nbuf = 1 - buf

        @pl.when(c + 1 < n_chunks)
        def _start_next():
            off = (c + 1) * CHUNK
            pltpu.async_copy(qk_hbm.at[pl.ds(off, CHUNK)], qkib.at[nbuf],
                             sem_q.at[nbuf])
            pltpu.async_copy(w_hbm.at[pl.ds(off, CHUNK)], wib.at[nbuf],
                             sem_w.at[nbuf])

        pltpu.make_async_copy(qk_hbm.at[pl.ds(0, CHUNK)], qkib.at[buf],
                              sem_q.at[buf]).wait()
        pltpu.make_async_copy(w_hbm.at[pl.ds(0, CHUNK)], wib.at[buf],
                              sem_w.at[buf]).wait()

        @plsc.parallel_loop(0, CHUNK // L, unroll=8)
        def accum_body(g):
            sl = pl.ds(g * L, L)
            qi, kvi = _unpack_qk(qkib[buf, sl])
            w = wib[buf, sl]
            plsc.addupdate_scatter(z_v, [qi], w)
            for j in range(D_PER // 2):
                pair = plsc.load_gather(kvp_v.at[pl.ds(j * NKV, NKV)], [kvi])
                c_even = plsc.bitcast(pair << 16, jnp.float32)
                c_odd = plsc.bitcast(
                    jnp.bitwise_and(pair, jnp.int32(-65536)), jnp.float32)
                plsc.addupdate_scatter(
                    acc_v.at[pl.ds((2 * j) * NQ, NQ)], [qi], w * c_even)
                plsc.addupdate_scatter(
                    acc_v.at[pl.ds((2 * j + 1) * NQ, NQ)], [qi], w * c_odd)

        return carry

    lax.fori_loop(0, n_chunks, chunk_body, 0)

    @plsc.parallel_loop(0, NQ // L, unroll=8)
    def scale_body(i):
        sl = pl.ds(i * L, L)
        r = 1.0 / (z_v[sl] + 1e-10)
        for d in range(D_PER):
            fsl = pl.ds(d * NQ + i * L, L)
            acc_v[fsl] = acc_v[fsl] * r

    for d in range(D_PER):
        pltpu.sync_copy(acc_v.at[pl.ds(d * NQ, NQ)],
                        acct_hbm.at[pl.ds((row0 + d) * NQ, NQ)])


def kernel(query_nodes, key_value_nodes, edge_index, proj_w, proj_b,
           attend_w, attend_b):
    kvt = key_value_nodes.T
    ab = jnp.reshape(attend_b, (1, 1))
    sq, skv, kvpp, qk = _project(
        query_nodes, kvt, edge_index,
        proj_w[0::2], proj_w[1::2], proj_b[0::2], proj_b[1::2],
        attend_w[0:D:2], attend_w[1:D:2], attend_w[D::2], attend_w[D + 1::2],
        ab)
    w = _sc_weights(qk, sq, skv)
    acct = _sc_aggregate(qk, w, kvpp.reshape(-1))
    return acct.reshape(D, NQ).T


# speedup vs baseline: 2.6269x; 1.0507x over previous
import functools

import jax
import jax.numpy as jnp
from jax import lax
from jax.experimental import pallas as pl
from jax.experimental.pallas import tpu as pltpu
from jax.experimental.pallas import tpu_sc as plsc

NQ = 10000
NKV = 10000
E = 320000
D = 128

NC = 2
NS = 16
NW = NC * NS
L = 16
D_PER = D // NW
CHUNK = 6400


def _tc_project(q_ref, kvt_ref, ei_ref, we_ref, wo_ref, be_ref, bo_ref,
                awqe_ref, awqo_ref, awkve_ref, awkvo_ref, ab_ref,
                sq_ref, skv_ref, kvpp_ref, qk_ref):
    kvt = kvt_ref[...]
    kvpt_e = jnp.dot(we_ref[...], kvt, preferred_element_type=jnp.float32)
    kvpt_e = kvpt_e + be_ref[...][:, None]
    kvpt_o = jnp.dot(wo_ref[...], kvt, preferred_element_type=jnp.float32)
    kvpt_o = kvpt_o + bo_ref[...][:, None]
    lo = jax.lax.bitcast_convert_type(
        kvpt_e.astype(jnp.bfloat16), jnp.uint16).astype(jnp.int32)
    hi = jax.lax.bitcast_convert_type(
        kvpt_o.astype(jnp.bfloat16), jnp.uint16).astype(jnp.int32)
    kvpp_ref[...] = lo | (hi << 16)
    v_q = (jnp.sum(awqe_ref[...][:, None] * we_ref[...], axis=0)
           + jnp.sum(awqo_ref[...][:, None] * wo_ref[...], axis=0))
    const = (jnp.sum(be_ref[...] * awqe_ref[...])
             + jnp.sum(bo_ref[...] * awqo_ref[...]) + ab_ref[0, 0])
    sq_ref[...] = jnp.sum(q_ref[...] * v_q[None, :], axis=1) + const
    skv_ref[...] = (jnp.sum(kvpt_e * awkve_ref[...][:, None], axis=0)
                    + jnp.sum(kvpt_o * awkvo_ref[...][:, None], axis=0))
    qk_ref[...] = ei_ref[0] * 65536 + ei_ref[1]


_project = pl.pallas_call(
    _tc_project,
    out_shape=[
        jax.ShapeDtypeStruct((NQ,), jnp.float32),
        jax.ShapeDtypeStruct((NKV,), jnp.float32),
        jax.ShapeDtypeStruct((D // 2, NKV), jnp.int32),
        jax.ShapeDtypeStruct((E,), jnp.int32),
    ],
)


def _unpack_qk(packed):
    qi = lax.shift_right_logical(packed, 16)
    kvi = jnp.bitwise_and(packed, 0xFFFF)
    return qi, kvi


_mesh = plsc.VectorSubcoreMesh(core_axis_name="c", subcore_axis_name="s")

E_PER = E // NW


@functools.partial(
    pl.kernel,
    out_type=[
        jax.ShapeDtypeStruct((E,), jnp.float32),
        jax.ShapeDtypeStruct((NW * NQ,), jnp.float32),
    ],
    mesh=_mesh,
    compiler_params=pltpu.CompilerParams(
        needs_layout_passes=False, disable_bounds_checks=True),
    scratch_types=[
        pltpu.VMEM((NQ,), jnp.float32),
        pltpu.VMEM((NKV,), jnp.float32),
        pltpu.VMEM((E_PER,), jnp.int32),
        pltpu.VMEM((E_PER,), jnp.float32),
        pltpu.VMEM((NQ,), jnp.float32),
    ],
)
def _sc_weights(qk_hbm, sq_hbm, skv_hbm, w_hbm, zp_hbm,
                sq_v, skv_v, qkb, wb, z_v):
    wid = lax.axis_index("s") * NC + lax.axis_index("c")
    base = wid * E_PER

    pltpu.sync_copy(sq_hbm, sq_v)
    pltpu.sync_copy(skv_hbm, skv_v)
    pltpu.sync_copy(qk_hbm.at[pl.ds(base, E_PER)], qkb)

    zeros = jnp.zeros((L,), jnp.float32)

    @plsc.parallel_loop(0, NQ // L, unroll=8)
    def zzero_body(i):
        z_v[pl.ds(i * L, L)] = zeros

    @plsc.parallel_loop(0, E_PER // L, unroll=8)
    def weight_body(g):
        sl = pl.ds(g * L, L)
        qi, kvi = _unpack_qk(qkb[sl])
        sq = plsc.load_gather(sq_v, [qi])
        skv = plsc.load_gather(skv_v, [kvi])
        e = sq + skv
        e = jnp.maximum(e, 0.2 * e)
        w = jnp.exp(e)
        wb[sl] = w
        plsc.addupdate_scatter(z_v, [qi], w)

    pltpu.sync_copy(wb, w_hbm.at[pl.ds(base, E_PER)])
    pltpu.sync_copy(z_v, zp_hbm.at[pl.ds(wid * NQ, NQ)])


@functools.partial(
    pl.kernel,
    out_type=jax.ShapeDtypeStruct((D * NQ,), jnp.float32),
    mesh=_mesh,
    compiler_params=pltpu.CompilerParams(
        needs_layout_passes=False, disable_bounds_checks=True),
    scratch_types=[
        pltpu.VMEM((D_PER // 2 * NKV,), jnp.int32),
        pltpu.VMEM((D_PER * NQ,), jnp.float32),
        pltpu.VMEM((2, CHUNK), jnp.int32),
        pltpu.VMEM((2, CHUNK), jnp.float32),
        pltpu.SemaphoreType.DMA((2,)),
        pltpu.SemaphoreType.DMA((2,)),
    ],
)
def _sc_aggregate(qk_hbm, w_hbm, kvpt_hbm, acct_hbm,
                  kvp_v, acc_v, qkib, wib, sem_q, sem_w):
    wid = lax.axis_index("s") * NC + lax.axis_index("c")
    row0 = wid * D_PER
    pair0 = wid * (D_PER // 2)

    for j in range(D_PER // 2):
        pltpu.sync_copy(kvpt_hbm.at[pl.ds((pair0 + j) * NKV, NKV)],
                        kvp_v.at[pl.ds(j * NKV, NKV)])

    zeros = jnp.zeros((L,), jnp.float32)

    @plsc.parallel_loop(0, NQ // L, unroll=8)
    def zero_body(i):
        for d in range(D_PER):
            acc_v[pl.ds(d * NQ + i * L, L)] = zeros

    n_chunks = E // CHUNK

    pltpu.async_copy(qk_hbm.at[pl.ds(0, CHUNK)], qkib.at[0], sem_q.at[0])
    pltpu.async_copy(w_hbm.at[pl.ds(0, CHUNK)], wib.at[0], sem_w.at[0])

    def chunk_body(c, carry):
        buf = lax.rem(c, 2)
        nbuf = 1 - buf

        @pl.when(c + 1 < n_chunks)
        def _start_next():
            off = (c + 1) * CHUNK
            pltpu.async_copy(qk_hbm.at[pl.ds(off, CHUNK)], qkib.at[nbuf],
                             sem_q.at[nbuf])
            pltpu.async_copy(w_hbm.at[pl.ds(off, CHUNK)], wib.at[nbuf],
                             sem_w.at[nbuf])

        pltpu.make_async_copy(qk_hbm.at[pl.ds(0, CHUNK)], qkib.at[buf],
                              sem_q.at[buf]).wait()
        pltpu.make_async_copy(w_hbm.at[pl.ds(0, CHUNK)], wib.at[buf],
                              sem_w.at[buf]).wait()

        @plsc.parallel_loop(0, CHUNK // L, unroll=8)
        def accum_body(g):
            sl = pl.ds(g * L, L)
            qi, kvi = _unpack_qk(qkib[buf, sl])
            w = wib[buf, sl]
            for j in range(D_PER // 2):
                pair = plsc.load_gather(kvp_v.at[pl.ds(j * NKV, NKV)], [kvi])
                c_even = plsc.bitcast(pair << 16, jnp.float32)
                c_odd = plsc.bitcast(
                    jnp.bitwise_and(pair, jnp.int32(-65536)), jnp.float32)
                plsc.addupdate_scatter(
                    acc_v.at[pl.ds((2 * j) * NQ, NQ)], [qi], w * c_even)
                plsc.addupdate_scatter(
                    acc_v.at[pl.ds((2 * j + 1) * NQ, NQ)], [qi], w * c_odd)

        return carry

    lax.fori_loop(0, n_chunks, chunk_body, 0)

    for d in range(D_PER):
        pltpu.sync_copy(acc_v.at[pl.ds(d * NQ, NQ)],
                        acct_hbm.at[pl.ds((row0 + d) * NQ, NQ)])


def _tc_scale(acct_ref, zp_ref, out_ref):
    z = jnp.sum(zp_ref[...], axis=0)
    out_ref[...] = acct_ref[...] * (1.0 / (z + 1e-10))[None, :]


_scale = pl.pallas_call(
    _tc_scale,
    out_shape=jax.ShapeDtypeStruct((D, NQ), jnp.float32),
)


def kernel(query_nodes, key_value_nodes, edge_index, proj_w, proj_b,
           attend_w, attend_b):
    kvt = key_value_nodes.T
    ab = jnp.reshape(attend_b, (1, 1))
    sq, skv, kvpp, qk = _project(
        query_nodes, kvt, edge_index,
        proj_w[0::2], proj_w[1::2], proj_b[0::2], proj_b[1::2],
        attend_w[0:D:2], attend_w[1:D:2], attend_w[D::2], attend_w[D + 1::2],
        ab)
    w, zp = _sc_weights(qk, sq, skv)
    acct = _sc_aggregate(qk, w, kvpp.reshape(-1))
    out_t = _scale(acct.reshape(D, NQ), zp.reshape(NW, NQ))
    return out_t.T


# speedup vs baseline: 2.6750x; 1.0183x over previous
import functools

import jax
import jax.numpy as jnp
from jax import lax
from jax.experimental import pallas as pl
from jax.experimental.pallas import tpu as pltpu
from jax.experimental.pallas import tpu_sc as plsc

NQ = 10000
NKV = 10000
E = 320000
D = 128

NC = 2
NS = 16
NW = NC * NS
L = 16
D_PER = D // NW
CHUNK = 6400


def _tc_project(q_ref, kvt_ref, ei_ref, we_ref, wo_ref, be_ref, bo_ref,
                awqe_ref, awqo_ref, awkve_ref, awkvo_ref, ab_ref,
                sq_ref, skv_ref, kvpp_ref, qk_ref):
    kvt = kvt_ref[...]
    kvpt_e = jnp.dot(we_ref[...], kvt, preferred_element_type=jnp.float32)
    kvpt_e = kvpt_e + be_ref[...][:, None]
    kvpt_o = jnp.dot(wo_ref[...], kvt, preferred_element_type=jnp.float32)
    kvpt_o = kvpt_o + bo_ref[...][:, None]
    lo = jax.lax.bitcast_convert_type(
        kvpt_e.astype(jnp.bfloat16), jnp.uint16).astype(jnp.int32)
    hi = jax.lax.bitcast_convert_type(
        kvpt_o.astype(jnp.bfloat16), jnp.uint16).astype(jnp.int32)
    kvpp_ref[...] = lo | (hi << 16)
    v_q = (jnp.sum(awqe_ref[...][:, None] * we_ref[...], axis=0)
           + jnp.sum(awqo_ref[...][:, None] * wo_ref[...], axis=0))
    const = (jnp.sum(be_ref[...] * awqe_ref[...])
             + jnp.sum(bo_ref[...] * awqo_ref[...]) + ab_ref[0, 0])
    sq_ref[...] = jnp.sum(q_ref[...] * v_q[None, :], axis=1) + const
    skv_ref[...] = (jnp.sum(kvpt_e * awkve_ref[...][:, None], axis=0)
                    + jnp.sum(kvpt_o * awkvo_ref[...][:, None], axis=0))
    qk_ref[...] = ei_ref[0] * 65536 + ei_ref[1]


_project = pl.pallas_call(
    _tc_project,
    out_shape=[
        jax.ShapeDtypeStruct((NQ,), jnp.float32),
        jax.ShapeDtypeStruct((NKV,), jnp.float32),
        jax.ShapeDtypeStruct((D // 2, NKV), jnp.int32),
        jax.ShapeDtypeStruct((E,), jnp.int32),
    ],
)


def _unpack_qk(packed):
    qi = lax.shift_right_logical(packed, 16)
    kvi = jnp.bitwise_and(packed, 0xFFFF)
    return qi, kvi


_mesh = plsc.VectorSubcoreMesh(core_axis_name="c", subcore_axis_name="s")

E_PER = E // NW


@functools.partial(
    pl.kernel,
    out_type=[
        jax.ShapeDtypeStruct((E,), jnp.float32),
        jax.ShapeDtypeStruct((NW * NQ,), jnp.float32),
    ],
    mesh=_mesh,
    compiler_params=pltpu.CompilerParams(
        needs_layout_passes=False, disable_bounds_checks=True),
    scratch_types=[
        pltpu.VMEM((NQ,), jnp.float32),
        pltpu.VMEM((NKV,), jnp.float32),
        pltpu.VMEM((E_PER,), jnp.int32),
        pltpu.VMEM((E_PER,), jnp.float32),
        pltpu.VMEM((NQ,), jnp.float32),
    ],
)
def _sc_weights(qk_hbm, sq_hbm, skv_hbm, w_hbm, zp_hbm,
                sq_v, skv_v, qkb, wb, z_v):
    wid = lax.axis_index("s") * NC + lax.axis_index("c")
    base = wid * E_PER

    pltpu.sync_copy(sq_hbm, sq_v)
    pltpu.sync_copy(skv_hbm, skv_v)
    pltpu.sync_copy(qk_hbm.at[pl.ds(base, E_PER)], qkb)

    zeros = jnp.zeros((L,), jnp.float32)

    @plsc.parallel_loop(0, NQ // L, unroll=8)
    def zzero_body(i):
        z_v[pl.ds(i * L, L)] = zeros

    @plsc.parallel_loop(0, E_PER // L, unroll=8)
    def weight_body(g):
        sl = pl.ds(g * L, L)
        qi, kvi = _unpack_qk(qkb[sl])
        sq = plsc.load_gather(sq_v, [qi])
        skv = plsc.load_gather(skv_v, [kvi])
        e = sq + skv
        e = jnp.maximum(e, 0.2 * e)
        w = jnp.exp(e)
        wb[sl] = w
        plsc.addupdate_scatter(z_v, [qi], w)

    pltpu.sync_copy(wb, w_hbm.at[pl.ds(base, E_PER)])
    pltpu.sync_copy(z_v, zp_hbm.at[pl.ds(wid * NQ, NQ)])


@functools.partial(
    pl.kernel,
    out_type=jax.ShapeDtypeStruct((D * NQ,), jnp.float32),
    mesh=_mesh,
    compiler_params=pltpu.CompilerParams(
        needs_layout_passes=False, disable_bounds_checks=True),
    scratch_types=[
        pltpu.VMEM((D_PER // 2 * NKV,), jnp.int32),
        pltpu.VMEM((D_PER * NQ,), jnp.float32),
        pltpu.VMEM((2, CHUNK), jnp.int32),
        pltpu.VMEM((2, CHUNK), jnp.float32),
        pltpu.SemaphoreType.DMA((2,)),
        pltpu.SemaphoreType.DMA((2,)),
    ],
)
def _sc_aggregate(qk_hbm, w_hbm, kvpt_hbm, acct_hbm,
                  kvp_v, acc_v, qkib, wib, sem_q, sem_w):
    wid = lax.axis_index("s") * NC + lax.axis_index("c")
    row0 = wid * D_PER
    pair0 = wid * (D_PER // 2)

    for j in range(D_PER // 2):
        pltpu.sync_copy(kvpt_hbm.at[pl.ds((pair0 + j) * NKV, NKV)],
                        kvp_v.at[pl.ds(j * NKV, NKV)])

    zeros = jnp.zeros((L,), jnp.float32)

    @plsc.parallel_loop(0, NQ // L, unroll=8)
    def zero_body(i):
        for d in range(D_PER):
            acc_v[pl.ds(d * NQ + i * L, L)] = zeros

    n_chunks = E // CHUNK

    pltpu.async_copy(qk_hbm.at[pl.ds(0, CHUNK)], qkib.at[0], sem_q.at[0])
    pltpu.async_copy(w_hbm.at[pl.ds(0, CHUNK)], wib.at[0], sem_w.at[0])

    def chunk_body(c, carry):
        buf = lax.rem(c, 2)
        nbuf = 1 - buf

        @pl.when(c + 1 < n_chunks)
        def _start_next():
            off = (c + 1) * CHUNK
            pltpu.async_copy(qk_hbm.at[pl.ds(off, CHUNK)], qkib.at[nbuf],
                             sem_q.at[nbuf])
            pltpu.async_copy(w_hbm.at[pl.ds(off, CHUNK)], wib.at[nbuf],
                             sem_w.at[nbuf])

        pltpu.make_async_copy(qk_hbm.at[pl.ds(0, CHUNK)], qkib.at[buf],
                              sem_q.at[buf]).wait()
        pltpu.make_async_copy(w_hbm.at[pl.ds(0, CHUNK)], wib.at[buf],
                              sem_w.at[buf]).wait()

        @plsc.parallel_loop(0, CHUNK // L, unroll=8)
        def accum_body(g):
            sl = pl.ds(g * L, L)
            qi, kvi = _unpack_qk(qkib[buf, sl])
            w = wib[buf, sl]
            for j in range(D_PER // 2):
                pair = plsc.load_gather(kvp_v.at[pl.ds(j * NKV, NKV)], [kvi])
                c_even = plsc.bitcast(pair << 16, jnp.float32)
                c_odd = plsc.bitcast(
                    jnp.bitwise_and(pair, jnp.int32(-65536)), jnp.float32)
                plsc.addupdate_scatter(
                    acc_v.at[pl.ds((2 * j) * NQ, NQ)], [qi], w * c_even)
                plsc.addupdate_scatter(
                    acc_v.at[pl.ds((2 * j + 1) * NQ, NQ)], [qi], w * c_odd)

        return carry

    lax.fori_loop(0, n_chunks, chunk_body, 0)

    for d in range(D_PER):
        pltpu.sync_copy(acc_v.at[pl.ds(d * NQ, NQ)],
                        acct_hbm.at[pl.ds((row0 + d) * NQ, NQ)])


def _tc_scale(acct_ref, zp_ref, out_ref):
    z = jnp.sum(zp_ref[...], axis=0)
    out_ref[...] = (acct_ref[...] * (1.0 / (z + 1e-10))[None, :]).T


_scale = pl.pallas_call(
    _tc_scale,
    out_shape=jax.ShapeDtypeStruct((NQ, D), jnp.float32),
)


def kernel(query_nodes, key_value_nodes, edge_index, proj_w, proj_b,
           attend_w, attend_b):
    kvt = key_value_nodes.T
    ab = jnp.reshape(attend_b, (1, 1))
    sq, skv, kvpp, qk = _project(
        query_nodes, kvt, edge_index,
        proj_w[0::2], proj_w[1::2], proj_b[0::2], proj_b[1::2],
        attend_w[0:D:2], attend_w[1:D:2], attend_w[D::2], attend_w[D + 1::2],
        ab)
    w, zp = _sc_weights(qk, sq, skv)
    acct = _sc_aggregate(qk, w, kvpp.reshape(-1))
    return _scale(acct.reshape(D, NQ), zp.reshape(NW, NQ))
